# baseline (speedup 1.0000x reference)
import functools

import jax
import jax.numpy as jnp
from jax import lax
from jax.experimental import pallas as pl
from jax.experimental.pallas import tpu as pltpu
from jax.experimental.pallas import tpu_sc as plsc

_NC = 2
_NS = 16
_NW = _NC * _NS
_CHUNK = 112
_NCHUNK = 2
_BPW = _CHUNK * _NCHUNK
_TOT = _NW * _BPW
_LANES = 16


def _sc_gather_body(flat_hbm, idx_hbm, mask_hbm, outv_hbm, outm_hbm,
                    idx_v, val_v, mask_v, vred_v, mred_v, sem):
    wid = lax.axis_index("s") * _NC + lax.axis_index("c")
    pltpu.sync_copy(idx_hbm.at[wid], idx_v)
    pltpu.sync_copy(mask_hbm.at[wid], mask_v)
    cps = [pltpu.async_copy(flat_hbm.at[idx_v.at[j]], val_v.at[j], sem)
           for j in range(_NCHUNK)]
    for cp in cps:
        cp.wait()
    accv = jnp.zeros((_LANES,), jnp.float32)
    accm = jnp.zeros((_LANES,), jnp.float32)
    for j in range(_NCHUNK):
        for k in range(_CHUNK // _LANES):
            v = val_v[j, pl.ds(k * _LANES, _LANES)]
            mk = mask_v[j, pl.ds(k * _LANES, _LANES)]
            accv = accv + v * mk
            accm = accm + mk
    vred_v[...] = accv
    mred_v[...] = accm
    pltpu.sync_copy(vred_v, outv_hbm.at[wid])
    pltpu.sync_copy(mred_v, outm_hbm.at[wid])


@functools.lru_cache(maxsize=1)
def _sc_gather():
    return pl.kernel(
        _sc_gather_body,
        out_type=[jax.ShapeDtypeStruct((_NW, _LANES), jnp.float32),
                  jax.ShapeDtypeStruct((_NW, _LANES), jnp.float32)],
        mesh=plsc.VectorSubcoreMesh(core_axis_name="c", subcore_axis_name="s",
                                    num_cores=_NC),
        scratch_types=[
            pltpu.VMEM((_NCHUNK, _CHUNK), jnp.int32),
            pltpu.VMEM((_NCHUNK, _CHUNK), jnp.float32),
            pltpu.VMEM((_NCHUNK, _CHUNK), jnp.float32),
            pltpu.VMEM((_LANES,), jnp.float32),
            pltpu.VMEM((_LANES,), jnp.float32),
            pltpu.SemaphoreType.DMA,
        ],
    )


def _tc_loss_body(a_ref, g_ref, m_ref, vp_ref, mp_ref, out_ref):
    m = m_ref[...].astype(jnp.float32)
    cnt = jnp.sum(m, axis=1, keepdims=True)
    n_m = jnp.sum(cnt)

    def masked_logsoftmax_sum(x):
        xmax = jnp.max(x, axis=1, keepdims=True)
        lse = jnp.log(jnp.sum(jnp.exp(x - xmax), axis=1, keepdims=True)) + xmax
        return jnp.sum(x * m) - jnp.sum(cnt * lse)

    out_ref[1] = -masked_logsoftmax_sum(a_ref[...]) / n_m
    out_ref[2] = -masked_logsoftmax_sum(g_ref[...]) / n_m
    out_ref[0] = -jnp.sum(vp_ref[...]) / jnp.sum(mp_ref[...])


def _tc_loss(a, g, m, vp, mp):
    return pl.pallas_call(
        _tc_loss_body,
        out_shape=jax.ShapeDtypeStruct((3,), jnp.float32),
        out_specs=pl.BlockSpec(memory_space=pltpu.SMEM),
    )(a, g, m, vp, mp)


def kernel(txt_input, att2_weights, ground_weights, target, att2_target,
           input_seq):
    b, s = target.shape
    n = b * s
    vocab = txt_input.shape[1]
    tgt = target.astype(jnp.int32)
    mask = jnp.concatenate(
        [jnp.ones((b, 1), dtype=bool), tgt[:, :-1] > 0], axis=1).reshape(-1)
    flat_idx = jnp.arange(n, dtype=jnp.int32) * vocab + tgt.reshape(-1)
    pad = _TOT - n
    idx_p = jnp.concatenate(
        [flat_idx, jnp.zeros((pad,), jnp.int32)]).reshape(_NW, _NCHUNK, _CHUNK)
    mask_p = jnp.concatenate(
        [mask.astype(jnp.float32),
         jnp.zeros((pad,), jnp.float32)]).reshape(_NW, _NCHUNK, _CHUNK)
    vp, mp = _sc_gather()(txt_input.reshape(-1), idx_p, mask_p)
    out = _tc_loss(att2_weights.reshape(n, -1), ground_weights.reshape(n, -1),
                   att2_target.reshape(n, -1),
                   vp.reshape(4, 128), mp.reshape(4, 128))
    return out[0], out[1], out[2]


# speedup vs baseline: 1.6751x; 1.6751x over previous
import functools

import jax
import jax.numpy as jnp
from jax import lax
from jax.experimental import pallas as pl
from jax.experimental.pallas import tpu as pltpu
from jax.experimental.pallas import tpu_sc as plsc

_NC = 2
_NS = 16
_NW = _NC * _NS
_LANES = 16
_GROUPS = 13
_BPW = _GROUPS * _LANES
_TOT = _NW * _BPW
_WIN = 128


def _sc_gather_body(txt_hbm, tgt_hbm, mask_hbm, outv_hbm, outm_hbm,
                    tgt_v, mask_v, buf_v, vred_v, mred_v, sem):
    n_rows = txt_hbm.shape[0]
    wid = lax.axis_index("s") * _NC + lax.axis_index("c")
    pltpu.sync_copy(tgt_hbm.at[wid], tgt_v)
    pltpu.sync_copy(mask_hbm.at[wid], mask_v)
    row0 = wid * _BPW

    def group(g, carry):
        accv, accm = carry
        tvec = tgt_v[pl.ds(g * _LANES, _LANES)]
        mvec = mask_v[pl.ds(g * _LANES, _LANES)]
        cps = []
        for l in range(_LANES):
            t_l = tvec[l]
            cb = pl.multiple_of(t_l & ~(_WIN - 1), _WIN)
            row = jnp.minimum(row0 + g * _LANES + l, n_rows - 1)
            rb = pl.multiple_of(row & ~7, 8)
            cps.append(pltpu.async_copy(
                txt_hbm.at[pl.ds(rb, 8), pl.ds(cb, _WIN)], buf_v.at[l], sem))
        for cp in cps:
            cp.wait()
        lane = lax.iota(jnp.int32, _LANES)
        rowvec = jnp.minimum(row0 + g * _LANES + lane, n_rows - 1)
        vals = plsc.load_gather(
            buf_v, [lane, rowvec & 7, tvec & (_WIN - 1)])
        return accv + vals * mvec, accm + mvec

    accv, accm = lax.fori_loop(
        0, _GROUPS, group,
        (jnp.zeros((_LANES,), jnp.float32), jnp.zeros((_LANES,), jnp.float32)))
    vred_v[...] = accv
    mred_v[...] = accm
    pltpu.sync_copy(vred_v, outv_hbm.at[wid])
    pltpu.sync_copy(mred_v, outm_hbm.at[wid])


@functools.lru_cache(maxsize=1)
def _sc_gather():
    return pl.kernel(
        _sc_gather_body,
        out_type=[jax.ShapeDtypeStruct((_NW, _LANES), jnp.float32),
                  jax.ShapeDtypeStruct((_NW, _LANES), jnp.float32)],
        mesh=plsc.VectorSubcoreMesh(core_axis_name="c", subcore_axis_name="s",
                                    num_cores=_NC),
        compiler_params=pltpu.CompilerParams(needs_layout_passes=False),
        scratch_types=[
            pltpu.VMEM((_BPW,), jnp.int32),
            pltpu.VMEM((_BPW,), jnp.float32),
            pltpu.VMEM((_LANES, 8, _WIN), jnp.float32),
            pltpu.VMEM((_LANES,), jnp.float32),
            pltpu.VMEM((_LANES,), jnp.float32),
            pltpu.SemaphoreType.DMA,
        ],
    )


def _tc_loss_body(a_ref, g_ref, m_ref, vp_ref, mp_ref, out_ref):
    m = m_ref[...].astype(jnp.float32)
    cnt = jnp.sum(m, axis=1, keepdims=True)
    n_m = jnp.sum(cnt)

    def masked_logsoftmax_sum(x):
        xmax = jnp.max(x, axis=1, keepdims=True)
        lse = jnp.log(jnp.sum(jnp.exp(x - xmax), axis=1, keepdims=True)) + xmax
        return jnp.sum(x * m) - jnp.sum(cnt * lse)

    out_ref[1] = -masked_logsoftmax_sum(a_ref[...]) / n_m
    out_ref[2] = -masked_logsoftmax_sum(g_ref[...]) / n_m
    out_ref[0] = -jnp.sum(vp_ref[...]) / jnp.sum(mp_ref[...])


def _tc_loss(a, g, m, vp, mp):
    return pl.pallas_call(
        _tc_loss_body,
        out_shape=jax.ShapeDtypeStruct((3,), jnp.float32),
        out_specs=pl.BlockSpec(memory_space=pltpu.SMEM),
    )(a, g, m, vp, mp)


def kernel(txt_input, att2_weights, ground_weights, target, att2_target,
           input_seq):
    b, s = target.shape
    n = b * s
    vocab = txt_input.shape[1]
    del vocab
    tgt = target.astype(jnp.int32)
    mask = jnp.concatenate(
        [jnp.ones((b, 1), dtype=bool), tgt[:, :-1] > 0], axis=1).reshape(-1)
    pad = _TOT - n
    tgt_p = jnp.concatenate(
        [tgt.reshape(-1), jnp.zeros((pad,), jnp.int32)]).reshape(_NW, _BPW)
    mask_p = jnp.concatenate(
        [mask.astype(jnp.float32),
         jnp.zeros((pad,), jnp.float32)]).reshape(_NW, _BPW)
    vp, mp = _sc_gather()(txt_input, tgt_p, mask_p)
    out = _tc_loss(att2_weights.reshape(n, -1), ground_weights.reshape(n, -1),
                   att2_target.reshape(n, -1),
                   vp.reshape(4, 128), mp.reshape(4, 128))
    return out[0], out[1], out[2]


# speedup vs baseline: 1.6752x; 1.0001x over previous
import functools

import jax
import jax.numpy as jnp
from jax import lax
from jax.experimental import pallas as pl
from jax.experimental.pallas import tpu as pltpu
from jax.experimental.pallas import tpu_sc as plsc

_NC = 2
_NS = 16
_NW = _NC * _NS
_LANES = 16
_GROUPS = 13
_BPW = _GROUPS * _LANES
_TOT = _NW * _BPW
_WIN = 128


def _sc_gather_body(txt_hbm, tgt_hbm, mask_hbm, outv_hbm, outm_hbm,
                    tgt_v, mask_v, buf_v, vred_v, mred_v, sem):
    n_rows = txt_hbm.shape[0]
    wid = lax.axis_index("s") * _NC + lax.axis_index("c")
    pltpu.sync_copy(tgt_hbm.at[wid], tgt_v)
    pltpu.sync_copy(mask_hbm.at[wid], mask_v)
    row0 = wid * _BPW
    lane = lax.iota(jnp.int32, _LANES)

    def group(g, carry):
        accv, accm = carry
        tvec = tgt_v[pl.ds(g * _LANES, _LANES)]
        mvec = mask_v[pl.ds(g * _LANES, _LANES)]
        tsc = [tvec[l] for l in range(_LANES)]
        cps = []
        for l in range(_LANES):
            t_l = tsc[l]
            cb = pl.multiple_of(t_l & ~(_WIN - 1), _WIN)
            row = jnp.minimum(row0 + g * _LANES + l, n_rows - 1)
            rb = pl.multiple_of(row & ~7, 8)
            cps.append(pltpu.async_copy(
                txt_hbm.at[pl.ds(rb, 8), pl.ds(cb, _WIN)], buf_v.at[l], sem))
        for l in range(_LANES):
            cps[l].wait()
            t_l = tsc[l]
            m_l = mvec[l]
            lane_sel = jnp.where(lane == (t_l & 15), m_l, 0.0)
            jt = (t_l >> 4) & 7
            for j in range(8):
                chunk = buf_v[l, l & 7, pl.ds(j * _LANES, _LANES)]
                accv = accv + chunk * jnp.where(jt == j, lane_sel, 0.0)
        return accv, accm + mvec

    accv, accm = lax.fori_loop(
        0, _GROUPS, group,
        (jnp.zeros((_LANES,), jnp.float32), jnp.zeros((_LANES,), jnp.float32)))
    vred_v[...] = accv
    mred_v[...] = accm
    pltpu.sync_copy(vred_v, outv_hbm.at[wid])
    pltpu.sync_copy(mred_v, outm_hbm.at[wid])


@functools.lru_cache(maxsize=1)
def _sc_gather():
    return pl.kernel(
        _sc_gather_body,
        out_type=[jax.ShapeDtypeStruct((_NW, _LANES), jnp.float32),
                  jax.ShapeDtypeStruct((_NW, _LANES), jnp.float32)],
        mesh=plsc.VectorSubcoreMesh(core_axis_name="c", subcore_axis_name="s",
                                    num_cores=_NC),
        scratch_types=[
            pltpu.VMEM((_BPW,), jnp.int32),
            pltpu.VMEM((_BPW,), jnp.float32),
            pltpu.VMEM((_LANES, 8, _WIN), jnp.float32),
            pltpu.VMEM((_LANES,), jnp.float32),
            pltpu.VMEM((_LANES,), jnp.float32),
            pltpu.SemaphoreType.DMA,
        ],
    )


def _tc_loss_body(a_ref, g_ref, m_ref, vp_ref, mp_ref, out_ref):
    m = m_ref[...].astype(jnp.float32)
    cnt = jnp.sum(m, axis=2, keepdims=True)
    n_m = jnp.sum(cnt)

    def masked_logsoftmax_sum(x):
        xmax = jnp.max(x, axis=2, keepdims=True)
        lse = jnp.log(jnp.sum(jnp.exp(x - xmax), axis=2, keepdims=True)) + xmax
        return jnp.sum(x * m) - jnp.sum(cnt * lse)

    out_ref[1] = -masked_logsoftmax_sum(a_ref[...]) / n_m
    out_ref[2] = -masked_logsoftmax_sum(g_ref[...]) / n_m
    out_ref[0] = -jnp.sum(vp_ref[...]) / jnp.sum(mp_ref[...])


def _tc_loss(a, g, m, vp, mp):
    return pl.pallas_call(
        _tc_loss_body,
        out_shape=jax.ShapeDtypeStruct((3,), jnp.float32),
        out_specs=pl.BlockSpec(memory_space=pltpu.SMEM),
    )(a, g, m, vp, mp)


def kernel(txt_input, att2_weights, ground_weights, target, att2_target,
           input_seq):
    b, s = target.shape
    n = b * s
    tgt = target.astype(jnp.int32)
    mask = jnp.concatenate(
        [jnp.ones((b, 1), dtype=bool), tgt[:, :-1] > 0], axis=1).reshape(-1)
    pad = _TOT - n
    tgt_p = jnp.concatenate(
        [tgt.reshape(-1), jnp.zeros((pad,), jnp.int32)]).reshape(_NW, _BPW)
    mask_p = jnp.concatenate(
        [mask.astype(jnp.float32),
         jnp.zeros((pad,), jnp.float32)]).reshape(_NW, _BPW)
    vp, mp = _sc_gather()(txt_input, tgt_p, mask_p)
    out = _tc_loss(att2_weights, ground_weights, att2_target, vp, mp)
    return out[0], out[1], out[2]


# speedup vs baseline: 10.5341x; 6.2883x over previous
import functools

import jax
import jax.numpy as jnp
from jax import lax
from jax.experimental import pallas as pl
from jax.experimental.pallas import tpu as pltpu
from jax.experimental.pallas import tpu_sc as plsc

_NC = 2
_NS = 16
_NW = _NC * _NS
_LANES = 16
_NACT = 25
_BPW = 256
_GROUPS = _BPW // _LANES
_SUB = 8
_OUTER = _GROUPS // _SUB


def _sc_gather_body(txt_hbm, tgt_hbm, mask_hbm, outv_hbm, outm_hbm,
                    tgt_v, mask_v, buf_a, buf_b, vred_v, mred_v,
                    sem_a, sem_b):
    wid = lax.axis_index("s") * _NC + lax.axis_index("c")
    lane = lax.iota(jnp.int32, _LANES)

    @pl.when(wid < _NACT)
    def _active():
        pltpu.sync_copy(tgt_hbm.at[wid], tgt_v)
        pltpu.sync_copy(mask_hbm.at[wid], mask_v)
        row0 = wid * _BPW
        bufs = (buf_a, buf_b)
        sems = (sem_a, sem_b)

        def outer(o, carry):
            accv, accm = carry
            goff = o * _SUB * _LANES
            cb = pl.multiple_of((row0 + goff) & ~127, 128)
            tvecs = [tgt_v[pl.ds(goff + gg * _LANES, _LANES)]
                     for gg in range(_SUB)]
            mvecs = [mask_v[pl.ds(goff + gg * _LANES, _LANES)]
                     for gg in range(_SUB)]
            tscs = [[tvecs[gg][l] for l in range(_LANES)]
                    for gg in range(_SUB)]

            def fire(gg):
                buf, sem = bufs[gg & 1], sems[gg & 1]
                return [pltpu.async_copy(
                    txt_hbm.at[pl.ds(pl.multiple_of(tscs[gg][l] & ~7, 8), 8),
                               pl.ds(cb, 128)],
                    buf.at[l], sem) for l in range(_LANES)]

            cps = {0: fire(0)}
            for gg in range(_SUB):
                if gg + 1 < _SUB:
                    cps[gg + 1] = fire(gg + 1)
                buf = bufs[gg & 1]
                for l in range(_LANES):
                    cps[gg][l].wait()
                    jt = tscs[gg][l] & 7
                    sel = jnp.where(lane == l, mvecs[gg][l], 0.0)
                    for s in range(8):
                        chunk = buf[l, s, pl.ds(gg * _LANES, _LANES)]
                        accv = accv + chunk * jnp.where(jt == s, sel, 0.0)
                accm = accm + mvecs[gg]
            return accv, accm

        accv, accm = lax.fori_loop(
            0, _OUTER, outer,
            (jnp.zeros((_LANES,), jnp.float32),
             jnp.zeros((_LANES,), jnp.float32)))
        vred_v[...] = accv
        mred_v[...] = accm
        pltpu.sync_copy(vred_v, outv_hbm.at[wid])
        pltpu.sync_copy(mred_v, outm_hbm.at[wid])


@functools.lru_cache(maxsize=1)
def _sc_gather():
    return pl.kernel(
        _sc_gather_body,
        out_type=[jax.ShapeDtypeStruct((_NACT, _LANES), jnp.float32),
                  jax.ShapeDtypeStruct((_NACT, _LANES), jnp.float32)],
        mesh=plsc.VectorSubcoreMesh(core_axis_name="c", subcore_axis_name="s",
                                    num_cores=_NC),
        scratch_types=[
            pltpu.VMEM((_BPW,), jnp.int32),
            pltpu.VMEM((_BPW,), jnp.float32),
            pltpu.VMEM((_LANES, 8, 128), jnp.float32),
            pltpu.VMEM((_LANES, 8, 128), jnp.float32),
            pltpu.VMEM((_LANES,), jnp.float32),
            pltpu.VMEM((_LANES,), jnp.float32),
            pltpu.SemaphoreType.DMA,
            pltpu.SemaphoreType.DMA,
        ],
    )


def _tc_soft_body(a_ref, g_ref, m_ref, out_ref):
    m = m_ref[...].astype(jnp.float32)
    cnt = jnp.sum(m, axis=1, keepdims=True)

    def masked_logsoftmax_sum(x):
        xmax = jnp.max(x, axis=1, keepdims=True)
        lse = jnp.log(jnp.sum(jnp.exp(x - xmax), axis=1, keepdims=True)) + xmax
        return jnp.sum(x * m) - jnp.sum(cnt * lse)

    out_ref[0] = masked_logsoftmax_sum(a_ref[...])
    out_ref[1] = masked_logsoftmax_sum(g_ref[...])
    out_ref[2] = jnp.sum(cnt)


def _tc_fin_body(s_ref, vp_ref, mp_ref, out_ref):
    n_m = s_ref[2]
    out_ref[0] = -jnp.sum(vp_ref[...]) / jnp.sum(mp_ref[...])
    out_ref[1] = -s_ref[0] / n_m
    out_ref[2] = -s_ref[1] / n_m


def _tc_soft(a, g, m):
    return pl.pallas_call(
        _tc_soft_body,
        out_shape=jax.ShapeDtypeStruct((3,), jnp.float32),
        out_specs=pl.BlockSpec(memory_space=pltpu.SMEM),
    )(a, g, m)


def _tc_fin(s, vp, mp):
    return pl.pallas_call(
        _tc_fin_body,
        in_specs=[pl.BlockSpec(memory_space=pltpu.SMEM),
                  pl.BlockSpec(memory_space=pltpu.VMEM),
                  pl.BlockSpec(memory_space=pltpu.VMEM)],
        out_shape=jax.ShapeDtypeStruct((3,), jnp.float32),
        out_specs=pl.BlockSpec(memory_space=pltpu.SMEM),
    )(s, vp, mp)


def kernel(txt_input, att2_weights, ground_weights, target, att2_target,
           input_seq):
    b, s = target.shape
    n = b * s
    tgt = target.astype(jnp.int32)
    mask = jnp.concatenate(
        [jnp.ones((b, 1), dtype=bool), tgt[:, :-1] > 0], axis=1).reshape(-1)
    tgt_p = tgt.reshape(n)[: _NACT * _BPW].reshape(_NACT, _BPW)
    mask_p = mask.astype(jnp.float32).reshape(_NACT, _BPW)
    vp, mp = _sc_gather()(txt_input.T, tgt_p, mask_p)
    at = jnp.transpose(att2_weights, (1, 2, 0))
    gt = jnp.transpose(ground_weights, (1, 2, 0))
    mt = jnp.transpose(att2_target, (1, 2, 0))
    sums = _tc_soft(at, gt, mt)
    out = _tc_fin(sums, vp, mp)
    return out[0], out[1], out[2]
